# Initial kernel scaffold; baseline (speedup 1.0000x reference)
import jax
import jax.numpy as jnp
from jax.experimental import pallas as pl


def kernel(class_logits, box_regression, proposals):
    raise NotImplementedError("write your pallas kernel here")


# speedup vs baseline: 15.2055x; 15.2055x over previous
import math

import jax
import jax.numpy as jnp
from jax import lax
from jax.experimental import pallas as pl
from jax.experimental.pallas import tpu as pltpu

_SCORE_THRESH = 0.05
_NMS_THRESH = 0.5
_DET_PER_IMG = 100
_MIN_SIZE = 1e-2
_BBOX_XFORM_CLIP = math.log(1000.0 / 16)
_IMG_H = 800.0
_IMG_W = 800.0
_N = 20000
_ROWS = 157
_PAD = _ROWS * 128


def _nms_body(l0, l1, dx, dy, dw, dh, px1, py1, px2, py2,
              coords_ref, score_ref, sm_ref):
    neg_inf = jnp.float32(-jnp.inf)

    row = lax.broadcasted_iota(jnp.int32, (_ROWS, 128), 0)
    col = lax.broadcasted_iota(jnp.int32, (_ROWS, 128), 1)
    idx = row * 128 + col
    inb = idx < _N

    m = jnp.maximum(l0[...], l1[...])
    e0 = jnp.exp(l0[...] - m)
    e1 = jnp.exp(l1[...] - m)
    s = e1 / (e0 + e1)

    w = px2[...] - px1[...]
    h = py2[...] - py1[...]
    cx = px1[...] + 0.5 * w
    cy = py1[...] + 0.5 * h
    tx = dx[...] / 10.0
    ty = dy[...] / 10.0
    tw = jnp.minimum(dw[...] / 5.0, _BBOX_XFORM_CLIP)
    th = jnp.minimum(dh[...] / 5.0, _BBOX_XFORM_CLIP)
    pcx = tx * w + cx
    pcy = ty * h + cy
    pw = jnp.exp(tw) * w
    ph = jnp.exp(th) * h
    x1 = pcx - 0.5 * pw
    y1 = pcy - 0.5 * ph
    x2 = pcx + 0.5 * pw
    y2 = pcy + 0.5 * ph
    x1 = jnp.clip(x1, 0.0, _IMG_W)
    x2 = jnp.clip(x2, 0.0, _IMG_W)
    y1 = jnp.clip(y1, 0.0, _IMG_H)
    y2 = jnp.clip(y2, 0.0, _IMG_H)

    ws = x2 - x1
    hs = y2 - y1
    valid = (s > _SCORE_THRESH) & (ws >= _MIN_SIZE) & (hs >= _MIN_SIZE) & inb

    coord_max = jnp.maximum(jnp.maximum(x1, y1), jnp.maximum(x2, y2))
    mc = jnp.max(jnp.where(valid, coord_max, neg_inf))
    off = mc + 1.0
    nx1 = x1 + off
    ny1 = y1 + off
    nx2 = x2 + off
    ny2 = y2 + off
    areas = (nx2 - nx1) * (ny2 - ny1)

    big = jnp.int32(2**30)
    lane = lax.broadcasted_iota(jnp.int32, (1, 128), 1)

    sm_ref[...] = jnp.where(valid, s, neg_inf)
    coords_ref[...] = jnp.zeros((4, 128), jnp.float32)
    score_ref[...] = jnp.zeros((1, 128), jnp.float32)

    def body(t, carry):
        smasked = sm_ref[...]
        mx = jnp.max(smasked)
        i = jnp.min(jnp.where(smasked == mx, idx, big))
        sel = idx == i
        xi1 = jnp.sum(jnp.where(sel, x1, 0.0))
        yi1 = jnp.sum(jnp.where(sel, y1, 0.0))
        xi2 = jnp.sum(jnp.where(sel, x2, 0.0))
        yi2 = jnp.sum(jnp.where(sel, y2, 0.0))
        si = jnp.sum(jnp.where(sel, s, 0.0))
        nxi1 = xi1 + off
        nyi1 = yi1 + off
        nxi2 = xi2 + off
        nyi2 = yi2 + off
        area_i = (nxi2 - nxi1) * (nyi2 - nyi1)
        xx1 = jnp.maximum(nxi1, nx1)
        yy1 = jnp.maximum(nyi1, ny1)
        xx2 = jnp.minimum(nxi2, nx2)
        yy2 = jnp.minimum(nyi2, ny2)
        iw = jnp.maximum(xx2 - xx1, 0.0)
        ih = jnp.maximum(yy2 - yy1, 0.0)
        inter = iw * ih
        iou = inter / (area_i + areas - inter)
        sm_ref[...] = jnp.where(iou <= _NMS_THRESH, smasked, neg_inf)
        tm = lane == t
        coords_ref[0:1, :] = jnp.where(tm, xi1, coords_ref[0:1, :])
        coords_ref[1:2, :] = jnp.where(tm, yi1, coords_ref[1:2, :])
        coords_ref[2:3, :] = jnp.where(tm, xi2, coords_ref[2:3, :])
        coords_ref[3:4, :] = jnp.where(tm, yi2, coords_ref[3:4, :])
        score_ref[...] = jnp.where(tm, si, score_ref[...])
        return carry

    lax.fori_loop(0, _DET_PER_IMG, body, 0)


def _pad2d(v):
    return jnp.pad(v, (0, _PAD - _N)).reshape(_ROWS, 128)


def kernel(class_logits, box_regression, proposals):
    args = [
        class_logits[:, 0], class_logits[:, 1],
        box_regression[:, 4], box_regression[:, 5],
        box_regression[:, 6], box_regression[:, 7],
        proposals[:, 0], proposals[:, 1],
        proposals[:, 2], proposals[:, 3],
    ]
    args = [_pad2d(a) for a in args]
    coords, score = pl.pallas_call(
        _nms_body,
        out_shape=(
            jax.ShapeDtypeStruct((4, 128), jnp.float32),
            jax.ShapeDtypeStruct((1, 128), jnp.float32),
        ),
        scratch_shapes=[pltpu.VMEM((_ROWS, 128), jnp.float32)],
    )(*args)
    boxes = coords[:, :_DET_PER_IMG].T
    scores = score[0, :_DET_PER_IMG]
    labels = jnp.ones((_DET_PER_IMG,), jnp.int32)
    return boxes, scores, labels
